# Initial kernel scaffold; baseline (speedup 1.0000x reference)
import jax
import jax.numpy as jnp
from jax.experimental import pallas as pl


def kernel(h, c, child_index, W_f, b_f, W_iou, b_iou):
    raise NotImplementedError("write your pallas kernel here")


# speedup vs baseline: 1.8647x; 1.8647x over previous
import functools

import jax
import jax.numpy as jnp
from jax import lax
from jax.experimental import pallas as pl
from jax.experimental.pallas import tpu as pltpu
from jax.experimental.pallas import tpu_sc as plsc

H = 128
NC, NS = 2, 16
NW = NC * NS
CH = 128
BP = 1024


def _sc_gather(h, c, idx2d):
    kpw = idx2d.shape[1]
    B = NW * kpw * CH
    mesh = plsc.VectorSubcoreMesh(core_axis_name="c", subcore_axis_name="s",
                                  num_cores=NC)

    @functools.partial(
        pl.kernel,
        out_type=[jax.ShapeDtypeStruct((B, H), jnp.float32),
                  jax.ShapeDtypeStruct((B, H), jnp.float32)],
        mesh=mesh,
        scratch_types=[
            pltpu.VMEM((kpw, CH), jnp.int32),
            pltpu.VMEM((2, CH, H), jnp.float32),
            pltpu.VMEM((2, CH, H), jnp.float32),
            pltpu.SemaphoreType.DMA,
            pltpu.SemaphoreType.DMA,
            pltpu.SemaphoreType.DMA((2,)),
            pltpu.SemaphoreType.DMA((2,)),
        ],
    )
    def gather_kernel(h_hbm, c_hbm, idx_hbm, hk_hbm, ck_hbm,
                      idx_v, bufh, bufc, gsh, gsc, wsh, wsc):
        wid = lax.axis_index("s") * NC + lax.axis_index("c")
        base_r = wid * kpw * CH

        pltpu.sync_copy(idx_hbm.at[wid], idx_v)

        def chunk(k, carry):
            slot = lax.rem(k, 2)
            r_here = base_r + k * CH

            @pl.when(k >= 2)
            def _():
                r_prev = base_r + (k - 2) * CH
                pltpu.make_async_copy(
                    bufh.at[slot], hk_hbm.at[pl.ds(r_prev, CH)],
                    wsh.at[slot]).wait()
                pltpu.make_async_copy(
                    bufc.at[slot], ck_hbm.at[pl.ds(r_prev, CH)],
                    wsc.at[slot]).wait()

            gh = pltpu.make_async_copy(h_hbm.at[idx_v.at[k]], bufh.at[slot], gsh)
            gc = pltpu.make_async_copy(c_hbm.at[idx_v.at[k]], bufc.at[slot], gsc)
            gh.start()
            gc.start()
            gh.wait()
            gc.wait()

            pltpu.make_async_copy(
                bufh.at[slot], hk_hbm.at[pl.ds(r_here, CH)], wsh.at[slot]).start()
            pltpu.make_async_copy(
                bufc.at[slot], ck_hbm.at[pl.ds(r_here, CH)], wsc.at[slot]).start()
            return carry

        lax.fori_loop(0, kpw, chunk, 0)

        for kk in (kpw - 2, kpw - 1):
            slot = kk % 2
            r0 = base_r + kk * CH
            pltpu.make_async_copy(
                bufh.at[slot], hk_hbm.at[pl.ds(r0, CH)], wsh.at[slot]).wait()
            pltpu.make_async_copy(
                bufc.at[slot], ck_hbm.at[pl.ds(r0, CH)], wsc.at[slot]).wait()

    return gather_kernel(h, c, idx2d)


def _tc_combine(hcat, ckk, W_f, b_f, W_iou, b_iou):
    P = hcat.shape[0]

    def body(hc_ref, ck_ref, wf_ref, bf_ref, wiou_ref, biou_ref, out_ref):
        hc = hc_ref[...]
        dn = (((1,), (1,)), ((), ()))
        f = jax.nn.sigmoid(
            lax.dot_general(hc, wf_ref[...], dn,
                            preferred_element_type=jnp.float32) + bf_ref[...])
        c_red = f[:, :H] * ck_ref[:, :H] + f[:, H:] * ck_ref[:, H:]
        iou = lax.dot_general(hc, wiou_ref[...], dn,
                              preferred_element_type=jnp.float32) + biou_ref[...]
        i_g = jax.nn.sigmoid(iou[:, :H])
        o_g = jax.nn.sigmoid(iou[:, H:2 * H])
        u = jnp.tanh(iou[:, 2 * H:])
        c_new = i_g * u + c_red
        h_new = o_g * jnp.tanh(c_new)
        out_ref[...] = jnp.concatenate([h_new, c_new], axis=1)

    return pl.pallas_call(
        body,
        grid=(P // BP,),
        in_specs=[
            pl.BlockSpec((BP, 2 * H), lambda i: (i, 0)),
            pl.BlockSpec((BP, 2 * H), lambda i: (i, 0)),
            pl.BlockSpec((2 * H, 2 * H), lambda i: (0, 0)),
            pl.BlockSpec((1, 2 * H), lambda i: (0, 0)),
            pl.BlockSpec((3 * H, 2 * H), lambda i: (0, 0)),
            pl.BlockSpec((1, 3 * H), lambda i: (0, 0)),
        ],
        out_specs=pl.BlockSpec((BP, 2 * H), lambda i: (i, 0)),
        out_shape=jax.ShapeDtypeStruct((P, 2 * H), jnp.float32),
    )(hcat, ckk, W_f, b_f.reshape(1, 2 * H), W_iou, b_iou)


def kernel(h, c, child_index, W_f, b_f, W_iou, b_iou):
    P = child_index.shape[0]
    idx = child_index.astype(jnp.int32).reshape(-1)
    B = idx.shape[0]
    kpw = -(-B // (NW * CH))
    B_pad = NW * kpw * CH
    idx = jnp.concatenate(
        [idx, jnp.zeros((B_pad - B,), jnp.int32)]).reshape(NW, kpw, CH)

    hk, ck = _sc_gather(h, c, idx)
    hcat = hk.reshape(B_pad // 2, 2 * H)
    ckk = ck.reshape(B_pad // 2, 2 * H)
    out = _tc_combine(hcat, ckk, W_f, b_f, W_iou, b_iou)
    return out[:P]


# speedup vs baseline: 2.8473x; 1.5269x over previous
import functools

import jax
import jax.numpy as jnp
from jax import lax
from jax.experimental import pallas as pl
from jax.experimental.pallas import tpu as pltpu
from jax.experimental.pallas import tpu_sc as plsc

H = 128
NC, NS = 2, 16
NW = NC * NS
CH = 128
BP = 1024


def _sc_gather(h, c, idx3d):
    kpw = idx3d.shape[1]
    B = NW * kpw * CH
    mesh = plsc.VectorSubcoreMesh(core_axis_name="c", subcore_axis_name="s",
                                  num_cores=NC)

    @functools.partial(
        pl.kernel,
        out_type=[jax.ShapeDtypeStruct((B, H), jnp.float32),
                  jax.ShapeDtypeStruct((B, H), jnp.float32)],
        mesh=mesh,
        scratch_types=[
            pltpu.VMEM((kpw, CH), jnp.int32),
            pltpu.VMEM((2, CH, H), jnp.float32),
            pltpu.VMEM((2, CH, H), jnp.float32),
            pltpu.SemaphoreType.DMA((2,)),
            pltpu.SemaphoreType.DMA((2,)),
            pltpu.SemaphoreType.DMA((2,)),
            pltpu.SemaphoreType.DMA((2,)),
        ],
    )
    def gather_kernel(h_hbm, c_hbm, idx_hbm, hk_hbm, ck_hbm,
                      idx_v, bufh, bufc, gsh, gsc, wsh, wsc):
        wid = lax.axis_index("s") * NC + lax.axis_index("c")
        base_r = wid * kpw * CH

        pltpu.sync_copy(idx_hbm.at[wid], idx_v)

        pltpu.make_async_copy(h_hbm.at[idx_v.at[0]], bufh.at[0],
                              gsh.at[0]).start()
        pltpu.make_async_copy(c_hbm.at[idx_v.at[0]], bufc.at[0],
                              gsc.at[0]).start()

        def chunk(k, carry):
            slot = lax.rem(k, 2)
            nslot = 1 - slot

            @pl.when(jnp.logical_and(k >= 1, k + 1 < kpw))
            def _():
                r_prev = base_r + (k - 1) * CH
                pltpu.make_async_copy(
                    bufh.at[nslot], hk_hbm.at[pl.ds(r_prev, CH)],
                    wsh.at[nslot]).wait()
                pltpu.make_async_copy(
                    bufc.at[nslot], ck_hbm.at[pl.ds(r_prev, CH)],
                    wsc.at[nslot]).wait()

            @pl.when(k + 1 < kpw)
            def _():
                pltpu.make_async_copy(h_hbm.at[idx_v.at[k + 1]],
                                      bufh.at[nslot], gsh.at[nslot]).start()
                pltpu.make_async_copy(c_hbm.at[idx_v.at[k + 1]],
                                      bufc.at[nslot], gsc.at[nslot]).start()

            r_here = base_r + k * CH
            pltpu.make_async_copy(h_hbm.at[idx_v.at[k]], bufh.at[slot],
                                  gsh.at[slot]).wait()
            pltpu.make_async_copy(c_hbm.at[idx_v.at[k]], bufc.at[slot],
                                  gsc.at[slot]).wait()
            pltpu.make_async_copy(
                bufh.at[slot], hk_hbm.at[pl.ds(r_here, CH)],
                wsh.at[slot]).start()
            pltpu.make_async_copy(
                bufc.at[slot], ck_hbm.at[pl.ds(r_here, CH)],
                wsc.at[slot]).start()
            return carry

        lax.fori_loop(0, kpw, chunk, 0)

        for kk in (kpw - 2, kpw - 1):
            slot = kk % 2
            r0 = base_r + kk * CH
            pltpu.make_async_copy(
                bufh.at[slot], hk_hbm.at[pl.ds(r0, CH)], wsh.at[slot]).wait()
            pltpu.make_async_copy(
                bufc.at[slot], ck_hbm.at[pl.ds(r0, CH)], wsc.at[slot]).wait()

    return gather_kernel(h, c, idx3d)


def _tc_combine(hk3, ck3, W_f, b_f, W_iou, b_iou):
    P = hk3.shape[1]

    def body(h0_ref, h1_ref, c0_ref, c1_ref, wf_ref, bf_ref, wiou_ref,
             biou_ref, out_ref):
        h0 = h0_ref[0]
        h1 = h1_ref[0]
        dn = (((1,), (1,)), ((), ()))
        wf = wf_ref[...]
        f = jax.nn.sigmoid(
            lax.dot_general(h0, wf[:, :H], dn,
                            preferred_element_type=jnp.float32)
            + lax.dot_general(h1, wf[:, H:], dn,
                              preferred_element_type=jnp.float32)
            + bf_ref[...])
        c_red = f[:, :H] * c0_ref[0] + f[:, H:] * c1_ref[0]
        wiou = wiou_ref[...]
        iou = (lax.dot_general(h0, wiou[:, :H], dn,
                               preferred_element_type=jnp.float32)
               + lax.dot_general(h1, wiou[:, H:], dn,
                                 preferred_element_type=jnp.float32)
               + biou_ref[...])
        i_g = jax.nn.sigmoid(iou[:, :H])
        o_g = jax.nn.sigmoid(iou[:, H:2 * H])
        u = jnp.tanh(iou[:, 2 * H:])
        c_new = i_g * u + c_red
        h_new = o_g * jnp.tanh(c_new)
        out_ref[...] = jnp.concatenate([h_new, c_new], axis=1)

    return pl.pallas_call(
        body,
        grid=(P // BP,),
        in_specs=[
            pl.BlockSpec((1, BP, H), lambda i: (0, i, 0)),
            pl.BlockSpec((1, BP, H), lambda i: (1, i, 0)),
            pl.BlockSpec((1, BP, H), lambda i: (0, i, 0)),
            pl.BlockSpec((1, BP, H), lambda i: (1, i, 0)),
            pl.BlockSpec((2 * H, 2 * H), lambda i: (0, 0)),
            pl.BlockSpec((1, 2 * H), lambda i: (0, 0)),
            pl.BlockSpec((3 * H, 2 * H), lambda i: (0, 0)),
            pl.BlockSpec((1, 3 * H), lambda i: (0, 0)),
        ],
        out_specs=pl.BlockSpec((BP, 2 * H), lambda i: (i, 0)),
        out_shape=jax.ShapeDtypeStruct((P, 2 * H), jnp.float32),
    )(hk3, hk3, ck3, ck3, W_f, b_f.reshape(1, 2 * H), W_iou, b_iou)


def kernel(h, c, child_index, W_f, b_f, W_iou, b_iou):
    P = child_index.shape[0]
    ci = child_index.astype(jnp.int32)
    kpw = -(-2 * P // (NW * CH))
    B_pad = NW * kpw * CH
    P_pad = B_pad // 2
    pad = jnp.zeros((P_pad - P,), jnp.int32)
    idx = jnp.concatenate([ci[:, 0], pad, ci[:, 1], pad]).reshape(NW, kpw, CH)

    hk, ck = _sc_gather(h, c, idx)
    hk3 = hk.reshape(2, P_pad, H)
    ck3 = ck.reshape(2, P_pad, H)
    out = _tc_combine(hk3, ck3, W_f, b_f, W_iou, b_iou)
    return out[:P]


# speedup vs baseline: 3.1815x; 1.1174x over previous
import functools

import jax
import jax.numpy as jnp
from jax import lax
from jax.experimental import pallas as pl
from jax.experimental.pallas import tpu as pltpu
from jax.experimental.pallas import tpu_sc as plsc

H = 128
NC, NS = 2, 16
NW = NC * NS
CH = 128
BP = 1000


def _sc_gather(h, c, idx3d):
    kpw = idx3d.shape[1]
    B = NW * kpw * CH
    mesh = plsc.VectorSubcoreMesh(core_axis_name="c", subcore_axis_name="s",
                                  num_cores=NC)

    @functools.partial(
        pl.kernel,
        out_type=[jax.ShapeDtypeStruct((B, H), jnp.float32),
                  jax.ShapeDtypeStruct((B, H), jnp.float32)],
        mesh=mesh,
        scratch_types=[
            pltpu.VMEM((kpw, CH), jnp.int32),
            pltpu.VMEM((2, CH, H), jnp.float32),
            pltpu.VMEM((2, CH, H), jnp.float32),
            pltpu.SemaphoreType.DMA((2,)),
            pltpu.SemaphoreType.DMA((2,)),
            pltpu.SemaphoreType.DMA((2,)),
            pltpu.SemaphoreType.DMA((2,)),
        ],
    )
    def gather_kernel(h_hbm, c_hbm, idx_hbm, hk_hbm, ck_hbm,
                      idx_v, bufh, bufc, gsh, gsc, wsh, wsc):
        wid = lax.axis_index("s") * NC + lax.axis_index("c")
        base_r = wid * kpw * CH

        pltpu.sync_copy(idx_hbm.at[wid], idx_v)

        pltpu.make_async_copy(h_hbm.at[idx_v.at[0]], bufh.at[0],
                              gsh.at[0]).start()
        pltpu.make_async_copy(c_hbm.at[idx_v.at[0]], bufc.at[0],
                              gsc.at[0]).start()

        def chunk(k, carry):
            slot = lax.rem(k, 2)
            nslot = 1 - slot

            @pl.when(jnp.logical_and(k >= 1, k + 1 < kpw))
            def _():
                r_prev = base_r + (k - 1) * CH
                pltpu.make_async_copy(
                    bufh.at[nslot], hk_hbm.at[pl.ds(r_prev, CH)],
                    wsh.at[nslot]).wait()
                pltpu.make_async_copy(
                    bufc.at[nslot], ck_hbm.at[pl.ds(r_prev, CH)],
                    wsc.at[nslot]).wait()

            @pl.when(k + 1 < kpw)
            def _():
                pltpu.make_async_copy(h_hbm.at[idx_v.at[k + 1]],
                                      bufh.at[nslot], gsh.at[nslot]).start()
                pltpu.make_async_copy(c_hbm.at[idx_v.at[k + 1]],
                                      bufc.at[nslot], gsc.at[nslot]).start()

            r_here = base_r + k * CH
            pltpu.make_async_copy(h_hbm.at[idx_v.at[k]], bufh.at[slot],
                                  gsh.at[slot]).wait()
            pltpu.make_async_copy(c_hbm.at[idx_v.at[k]], bufc.at[slot],
                                  gsc.at[slot]).wait()
            pltpu.make_async_copy(
                bufh.at[slot], hk_hbm.at[pl.ds(r_here, CH)],
                wsh.at[slot]).start()
            pltpu.make_async_copy(
                bufc.at[slot], ck_hbm.at[pl.ds(r_here, CH)],
                wsc.at[slot]).start()
            return carry

        lax.fori_loop(0, kpw, chunk, 0)

        for kk in (kpw - 2, kpw - 1):
            slot = kk % 2
            r0 = base_r + kk * CH
            pltpu.make_async_copy(
                bufh.at[slot], hk_hbm.at[pl.ds(r0, CH)], wsh.at[slot]).wait()
            pltpu.make_async_copy(
                bufc.at[slot], ck_hbm.at[pl.ds(r0, CH)], wsc.at[slot]).wait()

    return gather_kernel(h, c, idx3d)


def _tc_combine(P_out, hk3, ck3, W_f, b_f, W_iou, b_iou):
    def body(h0_ref, h1_ref, c0_ref, c1_ref, wf_ref, bf_ref, wiou_ref,
             biou_ref, out_ref):
        h0 = h0_ref[0]
        h1 = h1_ref[0]
        dn = (((1,), (1,)), ((), ()))
        wf = wf_ref[...]
        f = jax.nn.sigmoid(
            lax.dot_general(h0, wf[:, :H], dn,
                            preferred_element_type=jnp.float32)
            + lax.dot_general(h1, wf[:, H:], dn,
                              preferred_element_type=jnp.float32)
            + bf_ref[...])
        c_red = f[:, :H] * c0_ref[0] + f[:, H:] * c1_ref[0]
        wiou = wiou_ref[...]
        iou = (lax.dot_general(h0, wiou[:, :H], dn,
                               preferred_element_type=jnp.float32)
               + lax.dot_general(h1, wiou[:, H:], dn,
                                 preferred_element_type=jnp.float32)
               + biou_ref[...])
        i_g = jax.nn.sigmoid(iou[:, :H])
        o_g = jax.nn.sigmoid(iou[:, H:2 * H])
        u = jnp.tanh(iou[:, 2 * H:])
        c_new = i_g * u + c_red
        h_new = o_g * jnp.tanh(c_new)
        out_ref[...] = jnp.concatenate([h_new, c_new], axis=1)

    return pl.pallas_call(
        body,
        grid=(P_out // BP,),
        in_specs=[
            pl.BlockSpec((1, BP, H), lambda i: (0, i, 0)),
            pl.BlockSpec((1, BP, H), lambda i: (1, i, 0)),
            pl.BlockSpec((1, BP, H), lambda i: (0, i, 0)),
            pl.BlockSpec((1, BP, H), lambda i: (1, i, 0)),
            pl.BlockSpec((2 * H, 2 * H), lambda i: (0, 0)),
            pl.BlockSpec((1, 2 * H), lambda i: (0, 0)),
            pl.BlockSpec((3 * H, 2 * H), lambda i: (0, 0)),
            pl.BlockSpec((1, 3 * H), lambda i: (0, 0)),
        ],
        out_specs=pl.BlockSpec((BP, 2 * H), lambda i: (i, 0)),
        out_shape=jax.ShapeDtypeStruct((P_out, 2 * H), jnp.float32),
    )(hk3, hk3, ck3, ck3, W_f, b_f.reshape(1, 2 * H), W_iou, b_iou)


def kernel(h, c, child_index, W_f, b_f, W_iou, b_iou):
    P = child_index.shape[0]
    ci = child_index.astype(jnp.int32)
    kpw = -(-2 * P // (NW * CH))
    B_pad = NW * kpw * CH
    P_pad = B_pad // 2
    pad = jnp.zeros((P_pad - P,), jnp.int32)
    idx = jnp.concatenate([ci[:, 0], pad, ci[:, 1], pad]).reshape(NW, kpw, CH)

    hk, ck = _sc_gather(h, c, idx)
    hk3 = hk.reshape(2, P_pad, H)
    ck3 = ck.reshape(2, P_pad, H)
    return _tc_combine(P, hk3, ck3, W_f, b_f, W_iou, b_iou)
